# baseline (speedup 1.0000x reference)
import functools

import jax
import jax.numpy as jnp
from jax import lax
from jax.experimental import pallas as pl
from jax.experimental.pallas import tpu as pltpu
from jax.experimental.pallas import tpu_sc as plsc

NC = 2
NS = 16
NW = NC * NS
L = 16

T = 7840


def _make_kernel(n):
    assert n % L == 0
    nsub = -(-n // T)
    kpw = -(-nsub // NW)
    tail = n - (nsub - 1) * T
    assert tail % L == 0 and tail > 0

    mesh = plsc.VectorSubcoreMesh(
        core_axis_name="c", subcore_axis_name="s",
        num_cores=NC, num_subcores=NS)

    def body(ids_hbm, mp_hbm, out_hbm, in_buf, mp_buf, out_buf):
        wid = lax.axis_index("s") * NC + lax.axis_index("c")
        pltpu.sync_copy(mp_hbm, mp_buf)
        mpv = mp_buf[...]
        iota4 = lax.iota(jnp.int32, L) * 4

        def process(k, size):
            gbase = k * T

            @pl.when(k == 0)
            def _first():
                in_buf[pl.ds(0, L)] = jnp.full((L,), -1, jnp.int32)
                pltpu.sync_copy(ids_hbm.at[pl.ds(0, size)],
                                in_buf.at[pl.ds(8, size)])

            @pl.when(k != 0)
            def _rest():
                pltpu.sync_copy(ids_hbm.at[pl.ds(gbase - 8, size + 8)],
                                in_buf.at[pl.ds(0, size + 8)])

            def group(i, carry):
                cur = in_buf[pl.ds(8 + i * L, L)]
                prev = in_buf[pl.ds(7 + i * L, L)]
                one = jnp.full((L,), 1, jnp.int32)
                zero = jnp.full((L,), 0, jnp.int32)
                d0 = jnp.where(cur != prev, one, zero)
                d1 = jnp.where(cur == zero, one, zero)
                d3 = jnp.where(cur == mpv, one, zero)
                d2 = one - d1 - d3
                base = iota4 + jnp.full((L,), i * (4 * L), jnp.int32)
                plsc.store_scatter(out_buf, [base], d0)
                plsc.store_scatter(out_buf, [base + one], d1)
                plsc.store_scatter(out_buf, [base + one + one], d2)
                plsc.store_scatter(out_buf, [base + one + one + one], d3)
                return carry

            lax.fori_loop(0, size // L, group, 0)
            pltpu.sync_copy(out_buf.at[pl.ds(0, size * 4)],
                            out_hbm.at[pl.ds(gbase * 4, size * 4)])

        for j in range(kpw):
            k = wid * kpw + j

            @pl.when(k == nsub - 1)
            def _tail():
                process(k, tail)

            @pl.when(k < nsub - 1)
            def _full():
                process(k, T)

    kern = pl.kernel(
        body,
        out_type=jax.ShapeDtypeStruct((4 * n,), jnp.int32),
        mesh=mesh,
        compiler_params=pltpu.CompilerParams(needs_layout_passes=False),
        scratch_types=[
            pltpu.VMEM((T + 8,), jnp.int32),
            pltpu.VMEM((L,), jnp.int32),
            pltpu.VMEM((4 * T,), jnp.int32),
        ],
    )
    return kern


@jax.jit
def kernel(paragraph_doc_ids, max_paragraphs):
    ids = paragraph_doc_ids.astype(jnp.int32)
    n = ids.shape[0]
    mp_arr = jnp.full((L,), max_paragraphs - 1, jnp.int32)
    flat = _make_kernel(n)(ids, mp_arr)
    return flat.reshape(n, 4)


# speedup vs baseline: 1.0111x; 1.0111x over previous
import functools

import jax
import jax.numpy as jnp
from jax import lax
from jax.experimental import pallas as pl
from jax.experimental.pallas import tpu as pltpu
from jax.experimental.pallas import tpu_sc as plsc

NC = 2
NS = 16
NW = NC * NS
L = 16

T = 7840


def _make_kernel(n):
    assert n % L == 0
    nsub = -(-n // T)
    kpw = -(-nsub // NW)
    tail = n - (nsub - 1) * T
    assert tail % L == 0 and tail > 0

    mesh = plsc.VectorSubcoreMesh(
        core_axis_name="c", subcore_axis_name="s",
        num_cores=NC, num_subcores=NS)

    def body(ids_hbm, mp_hbm, out_hbm, in_buf, mp_buf, out_buf):
        wid = lax.axis_index("s") * NC + lax.axis_index("c")
        pltpu.sync_copy(mp_hbm, mp_buf)
        mpv = mp_buf[...]
        one = jnp.full((L,), 1, jnp.int32)
        zero = jnp.full((L,), 0, jnp.int32)
        idx0 = lax.iota(jnp.int32, L) * 4
        idx1 = idx0 + one
        idx2 = idx1 + one
        idx3 = idx2 + one

        def process(k, size):
            gbase = k * T

            @pl.when(k == 0)
            def _first():
                in_buf[pl.ds(0, L)] = jnp.full((L,), -1, jnp.int32)
                pltpu.sync_copy(ids_hbm.at[pl.ds(0, size)],
                                in_buf.at[pl.ds(8, size)])

            @pl.when(k != 0)
            def _rest():
                pltpu.sync_copy(ids_hbm.at[pl.ds(gbase - 8, size + 8)],
                                in_buf.at[pl.ds(0, size + 8)])

            @plsc.parallel_loop(0, size // L, unroll=8)
            def group(i):
                off = i * L
                cur = in_buf[pl.ds(8 + off, L)]
                prev = in_buf[pl.ds(7 + off, L)]
                d0 = jnp.where(cur != prev, one, zero)
                d1 = jnp.where(cur == zero, one, zero)
                d3 = jnp.where(cur == mpv, one, zero)
                d2 = one - d1 - d3
                ob = out_buf.at[pl.ds(off * 4, 4 * L)]
                plsc.store_scatter(ob, [idx0], d0)
                plsc.store_scatter(ob, [idx1], d1)
                plsc.store_scatter(ob, [idx2], d2)
                plsc.store_scatter(ob, [idx3], d3)
            pltpu.sync_copy(out_buf.at[pl.ds(0, size * 4)],
                            out_hbm.at[pl.ds(gbase * 4, size * 4)])

        for j in range(kpw):
            k = wid * kpw + j

            @pl.when(k == nsub - 1)
            def _tail():
                process(k, tail)

            @pl.when(k < nsub - 1)
            def _full():
                process(k, T)

    kern = pl.kernel(
        body,
        out_type=jax.ShapeDtypeStruct((4 * n,), jnp.int32),
        mesh=mesh,
        compiler_params=pltpu.CompilerParams(needs_layout_passes=False),
        scratch_types=[
            pltpu.VMEM((T + 8,), jnp.int32),
            pltpu.VMEM((L,), jnp.int32),
            pltpu.VMEM((4 * T,), jnp.int32),
        ],
    )
    return kern


@jax.jit
def kernel(paragraph_doc_ids, max_paragraphs):
    ids = paragraph_doc_ids.astype(jnp.int32)
    n = ids.shape[0]
    mp_arr = jnp.full((L,), max_paragraphs - 1, jnp.int32)
    flat = _make_kernel(n)(ids, mp_arr)
    return flat.reshape(n, 4)


# speedup vs baseline: 1.3943x; 1.3790x over previous
import functools

import jax
import jax.numpy as jnp
from jax import lax
from jax.experimental import pallas as pl
from jax.experimental.pallas import tpu as pltpu
from jax.experimental.pallas import tpu_sc as plsc

NC = 2
NS = 16
NW = NC * NS
L = 16

T = 7840


def _make_kernel(n):
    assert n % L == 0
    nsub = -(-n // T)
    kpw = -(-nsub // NW)
    tail = n - (nsub - 1) * T
    assert tail % L == 0 and tail > 0

    mesh = plsc.VectorSubcoreMesh(
        core_axis_name="c", subcore_axis_name="s",
        num_cores=NC, num_subcores=NS)

    def body(ids_hbm, mp_hbm, out_hbm, in_buf, mp_buf, out_buf):
        wid = lax.axis_index("s") * NC + lax.axis_index("c")
        pltpu.sync_copy(mp_hbm, mp_buf)
        mpv = mp_buf[...]
        one = jnp.full((L,), 1, jnp.int32)
        zero = jnp.full((L,), 0, jnp.int32)
        row = lax.iota(jnp.int32, L)
        col0 = zero
        col1 = one
        col2 = one + one
        col3 = col2 + one

        def process(k, size):
            gbase = k * T

            @pl.when(k == 0)
            def _first():
                in_buf[pl.ds(0, L)] = jnp.full((L,), -1, jnp.int32)
                pltpu.sync_copy(ids_hbm.at[pl.ds(0, size)],
                                in_buf.at[pl.ds(8, size)])

            @pl.when(k != 0)
            def _rest():
                pltpu.sync_copy(ids_hbm.at[pl.ds(gbase - 8, size + 8)],
                                in_buf.at[pl.ds(0, size + 8)])

            @plsc.parallel_loop(0, size // L, unroll=8)
            def group(i):
                off = i * L
                cur = in_buf[pl.ds(8 + off, L)]
                prev = in_buf[pl.ds(7 + off, L)]
                d0 = jnp.where(cur != prev, one, zero)
                d1 = jnp.where(cur == zero, one, zero)
                d3 = jnp.where(cur == mpv, one, zero)
                d2 = one - d1 - d3
                ob = out_buf.at[pl.ds(off, L)]
                plsc.store_scatter(ob, [row, col0], d0)
                plsc.store_scatter(ob, [row, col1], d1)
                plsc.store_scatter(ob, [row, col2], d2)
                plsc.store_scatter(ob, [row, col3], d3)
            pltpu.sync_copy(out_buf.at[pl.ds(0, size)],
                            out_hbm.at[pl.ds(gbase, size)])

        for j in range(kpw):
            k = wid * kpw + j

            @pl.when(k == nsub - 1)
            def _tail():
                process(k, tail)

            @pl.when(k < nsub - 1)
            def _full():
                process(k, T)

    kern = pl.kernel(
        body,
        out_type=jax.ShapeDtypeStruct((n, 4), jnp.int32),
        mesh=mesh,
        compiler_params=pltpu.CompilerParams(
            needs_layout_passes=False, use_tc_tiling_on_sc=False),
        scratch_types=[
            pltpu.VMEM((T + 8,), jnp.int32),
            pltpu.VMEM((L,), jnp.int32),
            pltpu.VMEM((T, 4), jnp.int32),
        ],
    )
    return kern


@jax.jit
def kernel(paragraph_doc_ids, max_paragraphs):
    ids = paragraph_doc_ids.astype(jnp.int32)
    n = ids.shape[0]
    mp_arr = jnp.full((L,), max_paragraphs - 1, jnp.int32)
    return _make_kernel(n)(ids, mp_arr)


# speedup vs baseline: 13.8624x; 9.9422x over previous
import functools

import jax
import jax.numpy as jnp
from jax import lax
from jax.experimental import pallas as pl
from jax.experimental.pallas import tpu as pltpu
from jax.experimental.pallas import tpu_sc as plsc

NC = 2
NS = 16
NW = NC * NS
L = 16

T = 7840


def _make_kernel(n):
    assert n % L == 0
    nsub = -(-n // T)
    kpw = -(-nsub // NW)
    tail = n - (nsub - 1) * T
    assert tail % L == 0 and tail > 0

    mesh = plsc.VectorSubcoreMesh(
        core_axis_name="c", subcore_axis_name="s",
        num_cores=NC, num_subcores=NS)

    def body(ids_hbm, mp_hbm, out_hbm, in_buf, mp_buf, out_buf):
        wid = lax.axis_index("s") * NC + lax.axis_index("c")
        pltpu.sync_copy(mp_hbm, mp_buf)
        mpv = mp_buf[...]
        one = jnp.full((L,), 1, jnp.int32)
        zero = jnp.full((L,), 0, jnp.int32)

        def process(k, size):
            gbase = k * T

            @pl.when(k == 0)
            def _first():
                in_buf[pl.ds(0, L)] = jnp.full((L,), -1, jnp.int32)
                pltpu.sync_copy(ids_hbm.at[pl.ds(0, size)],
                                in_buf.at[pl.ds(8, size)])

            @pl.when(k != 0)
            def _rest():
                pltpu.sync_copy(ids_hbm.at[pl.ds(gbase - 8, size + 8)],
                                in_buf.at[pl.ds(0, size + 8)])

            @plsc.parallel_loop(0, size // L, unroll=8)
            def group(i):
                off = i * L
                cur = in_buf[pl.ds(8 + off, L)]
                prev = in_buf[pl.ds(7 + off, L)]
                d0 = jnp.where(cur != prev, one, zero)
                d1 = jnp.where(cur == zero, one, zero)
                d3 = jnp.where(cur == mpv, one, zero)
                d2 = one - d1 - d3
                out_buf[0, pl.ds(off, L)] = d0
                out_buf[1, pl.ds(off, L)] = d1
                out_buf[2, pl.ds(off, L)] = d2
                out_buf[3, pl.ds(off, L)] = d3
            for f in range(4):
                pltpu.sync_copy(out_buf.at[f, pl.ds(0, size)],
                                out_hbm.at[f, pl.ds(gbase, size)])

        for j in range(kpw):
            k = wid * kpw + j

            @pl.when(k == nsub - 1)
            def _tail():
                process(k, tail)

            @pl.when(k < nsub - 1)
            def _full():
                process(k, T)

    kern = pl.kernel(
        body,
        out_type=jax.ShapeDtypeStruct((4, n), jnp.int32),
        mesh=mesh,
        compiler_params=pltpu.CompilerParams(
            needs_layout_passes=False, use_tc_tiling_on_sc=False),
        scratch_types=[
            pltpu.VMEM((T + 8,), jnp.int32),
            pltpu.VMEM((L,), jnp.int32),
            pltpu.VMEM((4, T), jnp.int32),
        ],
    )
    return kern


@jax.jit
def kernel(paragraph_doc_ids, max_paragraphs):
    ids = paragraph_doc_ids.astype(jnp.int32)
    n = ids.shape[0]
    mp_arr = jnp.full((L,), max_paragraphs - 1, jnp.int32)
    planes = _make_kernel(n)(ids, mp_arr)
    return planes.T


# speedup vs baseline: 15.6776x; 1.1309x over previous
import functools

import jax
import jax.numpy as jnp
from jax import lax
from jax.experimental import pallas as pl
from jax.experimental.pallas import tpu as pltpu
from jax.experimental.pallas import tpu_sc as plsc

NC = 2
NS = 16
NW = NC * NS
L = 16

T = 7840


def _make_kernel(n):
    assert n % L == 0
    nsub = -(-n // T)
    kpw = -(-nsub // NW)
    assert nsub == NW * kpw
    tail = n - (nsub - 1) * T
    assert 0 < tail <= T and tail % L == 0
    w_range = kpw * T

    mesh = plsc.VectorSubcoreMesh(
        core_axis_name="c", subcore_axis_name="s",
        num_cores=NC, num_subcores=NS)

    def body(ids_hbm, mp_hbm, out_hbm, in_buf, mp_buf, out_a, out_b,
             sem_a, sem_b):
        wid = lax.axis_index("s") * NC + lax.axis_index("c")
        pltpu.sync_copy(mp_hbm, mp_buf)
        mpv = mp_buf[...]
        one = jnp.full((L,), 1, jnp.int32)
        zero = jnp.full((L,), 0, jnp.int32)
        gbase0 = wid * w_range
        w_last = n - (NW - 1) * w_range

        @pl.when(wid == 0)
        def _head():
            in_buf[pl.ds(0, L)] = jnp.full((L,), -1, jnp.int32)
            pltpu.sync_copy(ids_hbm.at[pl.ds(0, w_range)],
                            in_buf.at[pl.ds(8, w_range)])

        @pl.when((wid != 0) & (wid != NW - 1))
        def _mid():
            pltpu.sync_copy(ids_hbm.at[pl.ds(gbase0 - 8, w_range + 8)],
                            in_buf.at[pl.ds(0, w_range + 8)])

        @pl.when(wid == NW - 1)
        def _last():
            pltpu.sync_copy(ids_hbm.at[pl.ds(gbase0 - 8, w_last + 8)],
                            in_buf.at[pl.ds(0, w_last + 8)])

        def compute(j, size, out_buf):
            local = j * T

            @plsc.parallel_loop(0, size // L, unroll=8)
            def group(i):
                off = i * L
                cur = in_buf[pl.ds(8 + local + off, L)]
                prev = in_buf[pl.ds(7 + local + off, L)]
                d0 = jnp.where(cur != prev, one, zero)
                d1 = jnp.where(cur == zero, one, zero)
                d3 = jnp.where(cur == mpv, one, zero)
                d2 = one - d1 - d3
                out_buf[0, pl.ds(off, L)] = d0
                out_buf[1, pl.ds(off, L)] = d1
                out_buf[2, pl.ds(off, L)] = d2
                out_buf[3, pl.ds(off, L)] = d3

        def plane_copies(j, size, out_buf, sem):
            gbase = gbase0 + j * T
            return [pltpu.make_async_copy(
                        out_buf.at[f, pl.ds(0, size)],
                        out_hbm.at[f, pl.ds(gbase, size)], sem)
                    for f in range(4)]

        bufs = (out_a, out_b)
        sems = (sem_a, sem_b)
        for j in range(kpw):
            out_buf, sem = bufs[j % 2], sems[j % 2]
            if j >= 2:
                for c in plane_copies(j - 2, T, out_buf, sem):
                    c.wait()
            is_tail = j == kpw - 1
            if not is_tail:
                compute(j, T, out_buf)
                for c in plane_copies(j, T, out_buf, sem):
                    c.start()
            else:
                k = wid * kpw + j

                @pl.when(k == nsub - 1)
                def _t():
                    compute(j, tail, out_buf)
                    for c in plane_copies(j, tail, out_buf, sem):
                        c.start()

                @pl.when(k != nsub - 1)
                def _f():
                    compute(j, T, out_buf)
                    for c in plane_copies(j, T, out_buf, sem):
                        c.start()

        for j in range(max(kpw - 2, 0), kpw):
            out_buf, sem = bufs[j % 2], sems[j % 2]
            if j != kpw - 1:
                for c in plane_copies(j, T, out_buf, sem):
                    c.wait()
            else:
                k = wid * kpw + j

                @pl.when(k == nsub - 1)
                def _tw():
                    for c in plane_copies(j, tail, out_buf, sem):
                        c.wait()

                @pl.when(k != nsub - 1)
                def _fw():
                    for c in plane_copies(j, T, out_buf, sem):
                        c.wait()

    kern = pl.kernel(
        body,
        out_type=jax.ShapeDtypeStruct((4, n), jnp.int32),
        mesh=mesh,
        compiler_params=pltpu.CompilerParams(
            needs_layout_passes=False, use_tc_tiling_on_sc=False),
        scratch_types=[
            pltpu.VMEM((w_range + 8,), jnp.int32),
            pltpu.VMEM((L,), jnp.int32),
            pltpu.VMEM((4, T), jnp.int32),
            pltpu.VMEM((4, T), jnp.int32),
            pltpu.SemaphoreType.DMA,
            pltpu.SemaphoreType.DMA,
        ],
    )
    return kern


@jax.jit
def kernel(paragraph_doc_ids, max_paragraphs):
    ids = paragraph_doc_ids.astype(jnp.int32)
    n = ids.shape[0]
    mp_arr = jnp.full((L,), max_paragraphs - 1, jnp.int32)
    planes = _make_kernel(n)(ids, mp_arr)
    return planes.T


# speedup vs baseline: 15.6951x; 1.0011x over previous
import functools

import jax
import jax.numpy as jnp
from jax import lax
from jax.experimental import pallas as pl
from jax.experimental.pallas import tpu as pltpu
from jax.experimental.pallas import tpu_sc as plsc

NC = 2
NS = 16
NW = NC * NS
L = 16

T = 7840


def _make_kernel(n):
    assert n % L == 0
    nsub = -(-n // T)
    kpw = -(-nsub // NW)
    assert nsub == NW * kpw
    tail = n - (nsub - 1) * T
    assert 0 < tail <= T and tail % L == 0
    w_range = kpw * T

    mesh = plsc.VectorSubcoreMesh(
        core_axis_name="c", subcore_axis_name="s",
        num_cores=NC, num_subcores=NS)

    def body(ids_hbm, mp_hbm, out_hbm, in_buf, mp_buf, out_a, out_b,
             sem_a, sem_b):
        wid = lax.axis_index("s") * NC + lax.axis_index("c")
        pltpu.sync_copy(mp_hbm, mp_buf)
        mpv = mp_buf[...]
        one = jnp.full((L,), 1, jnp.int32)
        zero = jnp.full((L,), 0, jnp.int32)
        gbase0 = wid * w_range
        w_last = n - (NW - 1) * w_range

        @pl.when(wid == 0)
        def _head():
            in_buf[pl.ds(0, L)] = jnp.full((L,), -1, jnp.int32)
            pltpu.sync_copy(ids_hbm.at[pl.ds(0, w_range)],
                            in_buf.at[pl.ds(8, w_range)])

        @pl.when((wid != 0) & (wid != NW - 1))
        def _mid():
            pltpu.sync_copy(ids_hbm.at[pl.ds(gbase0 - 8, w_range + 8)],
                            in_buf.at[pl.ds(0, w_range + 8)])

        @pl.when(wid == NW - 1)
        def _last():
            pltpu.sync_copy(ids_hbm.at[pl.ds(gbase0 - 8, w_last + 8)],
                            in_buf.at[pl.ds(0, w_last + 8)])

        def compute(j, size, out_buf):
            local = j * T

            @plsc.parallel_loop(0, size // L, unroll=8)
            def group(i):
                off = i * L
                cur = in_buf[pl.ds(8 + local + off, L)]
                prev = in_buf[pl.ds(7 + local + off, L)]
                d0 = jnp.where(cur != prev, one, zero)
                d1 = jnp.where(cur == zero, one, zero)
                d3 = jnp.where(cur == mpv, one, zero)
                d2 = one - d1 - d3
                out_buf[0, pl.ds(off, L)] = d0
                out_buf[1, pl.ds(off, L)] = d1
                out_buf[2, pl.ds(off, L)] = d2
                out_buf[3, pl.ds(off, L)] = d3

        def plane_copies(j, size, out_buf, sem):
            gbase = gbase0 + j * T
            return [pltpu.make_async_copy(
                        out_buf.at[f, pl.ds(0, size)],
                        out_hbm.at[f, pl.ds(gbase, size)], sem)
                    for f in range(4)]

        bufs = (out_a, out_b)
        sems = (sem_a, sem_b)
        for j in range(kpw):
            out_buf, sem = bufs[j % 2], sems[j % 2]
            if j >= 2:
                for c in plane_copies(j - 2, T, out_buf, sem):
                    c.wait()
            is_tail = j == kpw - 1
            if not is_tail:
                compute(j, T, out_buf)
                for c in plane_copies(j, T, out_buf, sem):
                    c.start()
            else:
                k = wid * kpw + j

                @pl.when(k == nsub - 1)
                def _t():
                    compute(j, tail, out_buf)
                    for c in plane_copies(j, tail, out_buf, sem):
                        c.start()

                @pl.when(k != nsub - 1)
                def _f():
                    compute(j, T, out_buf)
                    for c in plane_copies(j, T, out_buf, sem):
                        c.start()

        for j in range(max(kpw - 2, 0), kpw):
            out_buf, sem = bufs[j % 2], sems[j % 2]
            if j != kpw - 1:
                for c in plane_copies(j, T, out_buf, sem):
                    c.wait()
            else:
                k = wid * kpw + j

                @pl.when(k == nsub - 1)
                def _tw():
                    for c in plane_copies(j, tail, out_buf, sem):
                        c.wait()

                @pl.when(k != nsub - 1)
                def _fw():
                    for c in plane_copies(j, T, out_buf, sem):
                        c.wait()

    kern = pl.kernel(
        body,
        out_type=jax.ShapeDtypeStruct((4, n), jnp.int32),
        mesh=mesh,
        compiler_params=pltpu.CompilerParams(
            needs_layout_passes=False, use_tc_tiling_on_sc=False,
            skip_device_barrier=True),
        scratch_types=[
            pltpu.VMEM((w_range + 8,), jnp.int32),
            pltpu.VMEM((L,), jnp.int32),
            pltpu.VMEM((4, T), jnp.int32),
            pltpu.VMEM((4, T), jnp.int32),
            pltpu.SemaphoreType.DMA,
            pltpu.SemaphoreType.DMA,
        ],
    )
    return kern


@jax.jit
def kernel(paragraph_doc_ids, max_paragraphs):
    ids = paragraph_doc_ids.astype(jnp.int32)
    n = ids.shape[0]
    mp_arr = jnp.full((L,), max_paragraphs - 1, jnp.int32)
    planes = _make_kernel(n)(ids, mp_arr)
    return planes.T
